# baseline (speedup 1.0000x reference)
import functools

import jax
import jax.numpy as jnp
from jax import lax
from jax.experimental import pallas as pl
from jax.experimental.pallas import tpu as pltpu
from jax.experimental.pallas import tpu_sc as plsc

MARGIN = 0.0
EPS = 1e-08
SELECT_RATIO = 30

N_TOTAL = 12_800_000
ROWS = 1000
COLS = 12_800
BLK_ROWS = 8

NC = 2
NS = 16
NW = NC * NS
CHUNK = N_TOTAL // NW
W = 2000
NWIN = CHUNK // W
NVREG = W // 16
PAD = NW * W
NBINS = 2048
SHIFTS = (0, 11, 22)

_mesh = plsc.VectorSubcoreMesh(core_axis_name="c", subcore_axis_name="s")
_params = pltpu.CompilerParams(needs_layout_passes=False)


def _elemwise_body(inp_ref, tgt_ref, val_ref, kw_ref):
    x = inp_ref[...]
    t = tgt_ref[...]
    sig = jax.nn.sigmoid(x)
    w = 1.0 - sig
    m = t > MARGIN
    vpos = -jnp.log(jnp.clip(sig, EPS, 1.0 - EPS))
    vneg = -jnp.log(jnp.maximum(w, EPS))
    val_ref[...] = jnp.where(m, vpos, vneg)
    logw = jnp.log(jnp.maximum(w, 1e-30))
    kw_ref[...] = jnp.where(m, 1.0, logw)


def _elemwise(inp_flat, tgt_flat):
    inp2 = inp_flat.reshape(ROWS, COLS)
    tgt2 = tgt_flat.reshape(ROWS, COLS)
    val, kw = pl.pallas_call(
        _elemwise_body,
        grid=(ROWS // BLK_ROWS,),
        in_specs=[
            pl.BlockSpec((BLK_ROWS, COLS), lambda i: (i, 0)),
            pl.BlockSpec((BLK_ROWS, COLS), lambda i: (i, 0)),
        ],
        out_specs=[
            pl.BlockSpec((BLK_ROWS, COLS), lambda i: (i, 0)),
            pl.BlockSpec((BLK_ROWS, COLS), lambda i: (i, 0)),
        ],
        out_shape=[
            jax.ShapeDtypeStruct((ROWS, COLS), jnp.float32),
            jax.ShapeDtypeStruct((ROWS, COLS), jnp.float32),
        ],
    )(inp2, tgt2)
    return val.reshape(-1), kw.reshape(-1)


def _wid():
    return lax.axis_index("s") * NC + lax.axis_index("c")


def _iota16():
    return lax.iota(jnp.int32, 16)


def _splat(x):
    return jnp.full((16,), x, jnp.int32)


@functools.partial(
    pl.kernel, mesh=_mesh, compiler_params=_params,
    out_type=jax.ShapeDtypeStruct((NW, 16), jnp.int32),
    scratch_types=[pltpu.VMEM((W,), jnp.float32),
                   pltpu.VMEM((16,), jnp.int32),
                   pltpu.SemaphoreType.DMA],
)
def _sc_count(tgt_hbm, counts_hbm, buf_v, cnt_v, sem):
    wid = _wid()
    base = wid * CHUNK

    def win(k, acc):
        pltpu.sync_copy(tgt_hbm.at[pl.ds(base + k * W, W)], buf_v)

        def vstep(j, a):
            t = buf_v[pl.ds(j * 16, 16)]
            return a + jnp.where(t > MARGIN, 1, 0).astype(jnp.int32)

        return lax.fori_loop(0, NVREG, vstep, acc)

    acc = lax.fori_loop(0, NWIN, win, jnp.zeros((16,), jnp.int32))
    cnt_v[...] = _splat(jnp.sum(acc))
    pltpu.sync_copy(cnt_v, counts_hbm.at[wid])


def _load_counts(counts_hbm, counts_v, wid):
    pltpu.sync_copy(counts_hbm, counts_v)
    psum = jnp.int32(0)
    total = jnp.int32(0)
    for i in range(NW):
        v = jnp.max(counts_v[i, :])
        psum = psum + jnp.where(i < wid, v, 0)
        total = total + v
    return psum, total


@functools.partial(
    pl.kernel, mesh=_mesh, compiler_params=_params,
    out_type=[jax.ShapeDtypeStruct((N_TOTAL + PAD,), jnp.float32),
              jax.ShapeDtypeStruct((N_TOTAL + PAD,), jnp.int32),
              jax.ShapeDtypeStruct((N_TOTAL + PAD,), jnp.float32)],
    scratch_types=[pltpu.VMEM((NW, 16), jnp.int32),
                   pltpu.VMEM((W,), jnp.float32),
                   pltpu.VMEM((W,), jnp.float32),
                   pltpu.VMEM((W,), jnp.int32),
                   pltpu.VMEM((W,), jnp.int32),
                   pltpu.VMEM((W,), jnp.float32),
                   pltpu.VMEM((W,), jnp.int32),
                   pltpu.SemaphoreType.DMA],
)
def _sc_compact(counts_hbm, kw_hbm, val_hbm, g_hbm,
                out_hbm, ku_hbm, nv_hbm,
                counts_v, kwv, valv, oidx_v, nidx_v, gv, kuv, sem):
    wid = _wid()
    pos_base, _total = _load_counts(counts_hbm, counts_v, wid)
    neg_base = wid * CHUNK - pos_base
    base = wid * CHUNK
    iota = _iota16()
    dump = N_TOTAL + wid * W

    def win(k, carry):
        pr, nr = carry
        pltpu.sync_copy(kw_hbm.at[pl.ds(base + k * W, W)], kwv)
        pltpu.sync_copy(val_hbm.at[pl.ds(base + k * W, W)], valv)

        def vstep(j, c):
            pr2, nr2 = c
            kvec = kwv[pl.ds(j * 16, 16)]
            m = kvec > 0.0
            mi = m.astype(jnp.int32)
            pc = plsc.cumsum(mi)
            nc = plsc.cumsum(1 - mi)
            wpos = j * 16 + iota
            oidx_v[pl.ds(j * 16, 16)] = jnp.where(m, pr2 + pc - 1, dump + wpos)
            nidx_v[pl.ds(j * 16, 16)] = jnp.where(m, dump + wpos, nr2 + nc - 1)
            return (pr2 + jnp.sum(mi), nr2 + jnp.sum(1 - mi))

        pr, nr = lax.fori_loop(0, NVREG, vstep, (pr, nr))
        pltpu.async_copy(g_hbm.at[nidx_v], gv, sem).wait()

        def vstep2(j, c):
            kvec = kwv[pl.ds(j * 16, 16)]
            key = kvec + gv[pl.ds(j * 16, 16)]
            bits = plsc.bitcast(key, jnp.int32)
            ku = jnp.where(bits < 0, bits, bits ^ jnp.int32(0x7FFFFFFF))
            kuv[pl.ds(j * 16, 16)] = ku
            return c

        lax.fori_loop(0, NVREG, vstep2, 0)
        cp1 = pltpu.async_copy(valv, out_hbm.at[oidx_v], sem)
        cp2 = pltpu.async_copy(kuv, ku_hbm.at[nidx_v], sem)
        cp3 = pltpu.async_copy(valv, nv_hbm.at[nidx_v], sem)
        cp1.wait()
        cp2.wait()
        cp3.wait()
        return (pr, nr)

    lax.fori_loop(0, NWIN, win, (pos_base, neg_base))


def _hist_body(shift, counts_hbm, ku_hbm, hist_hbm, counts_v, kuv, hist_v, sem):
    wid = _wid()
    _pb, total = _load_counts(counts_hbm, counts_v, wid)
    nneg = N_TOTAL - total
    nwin_total = (nneg + (W - 1)) // W
    share = (nwin_total + (NW - 1)) // NW
    iota = _iota16()

    def zstep(j, c):
        hist_v[pl.ds(j * 16, 16)] = jnp.zeros((16,), jnp.int32)
        return c

    lax.fori_loop(0, NBINS // 16, zstep, 0)

    def win(t, c):
        gbase = (wid * share + t) * W

        @pl.when(gbase < nneg)
        def _():
            pltpu.sync_copy(ku_hbm.at[pl.ds(gbase, W)], kuv)

            def vstep(j, c2):
                ku = kuv[pl.ds(j * 16, 16)]
                gpos = gbase + j * 16 + iota
                m = gpos < nneg
                d = lax.shift_right_logical(ku, shift) & (NBINS - 1)
                occ, lastm = plsc.scan_count(d, mask=m)
                plsc.addupdate_scatter(hist_v, [d], occ, mask=lastm)
                return c2

            lax.fori_loop(0, NVREG, vstep, 0)

        return c

    lax.fori_loop(0, share, win, 0)
    pltpu.sync_copy(hist_v, hist_hbm.at[wid])


def _perm_body(shift, is_final,
               counts_hbm, hist_hbm, ku_hbm, nv_hbm, *refs):
    if is_final:
        (outn_hbm, counts_v, histf_v, offs_v, kuv, vav, didx_v, dat_v,
         sem) = refs
    else:
        (outk_hbm, outv_hbm, counts_v, histf_v, offs_v, kuv, vav, didx_v,
         sem) = refs
    wid = _wid()
    pos_base, total = _load_counts(counts_hbm, counts_v, wid)
    nneg = N_TOTAL - total
    nsamp = jnp.minimum(SELECT_RATIO * total, nneg)
    nwin_total = (nneg + (W - 1)) // W
    share = (nwin_total + (NW - 1)) // NW
    iota = _iota16()
    dump = N_TOTAL + wid * W

    pltpu.sync_copy(hist_hbm, histf_v)

    def dgroup(jg, runtot):
        zero = jnp.zeros((16,), jnp.int32)

        def wrow(i, cp):
            cs, ps = cp
            row = histf_v[i, pl.ds(jg * 16, 16)]
            return (cs + row, ps + jnp.where(_splat(i) < wid, row, 0))

        colsum, presum = lax.fori_loop(0, NW, wrow, (zero, zero))
        excl = plsc.cumsum(colsum) - colsum
        offs_v[pl.ds(jg * 16, 16)] = runtot + excl + presum
        return runtot + jnp.sum(colsum)

    lax.fori_loop(0, NBINS // 16, dgroup, jnp.int32(0))

    def win(t, c):
        gbase = (wid * share + t) * W

        @pl.when(gbase < nneg)
        def _():
            pltpu.sync_copy(ku_hbm.at[pl.ds(gbase, W)], kuv)
            pltpu.sync_copy(nv_hbm.at[pl.ds(gbase, W)], vav)

            def vstep(j, c2):
                ku = kuv[pl.ds(j * 16, 16)]
                gpos = gbase + j * 16 + iota
                m = gpos < nneg
                d = lax.shift_right_logical(ku, shift) & (NBINS - 1)
                occ, lastm = plsc.scan_count(d, mask=m)
                cur = plsc.load_gather(offs_v, [d])
                pos = cur + occ - 1
                plsc.addupdate_scatter(offs_v, [d], occ, mask=lastm)
                wpos = j * 16 + iota
                if is_final:
                    didx_v[pl.ds(j * 16, 16)] = jnp.where(
                        m, total + pos, dump + wpos)
                    va = vav[pl.ds(j * 16, 16)]
                    dat_v[pl.ds(j * 16, 16)] = jnp.where(
                        jnp.logical_and(m, pos < nsamp), va, 0.0)
                else:
                    didx_v[pl.ds(j * 16, 16)] = jnp.where(m, pos, dump + wpos)
                return c2

            lax.fori_loop(0, NVREG, vstep, 0)
            if is_final:
                pltpu.async_copy(dat_v, outn_hbm.at[didx_v], sem).wait()
            else:
                cp1 = pltpu.async_copy(kuv, outk_hbm.at[didx_v], sem)
                cp2 = pltpu.async_copy(vav, outv_hbm.at[didx_v], sem)
                cp1.wait()
                cp2.wait()

        return c

    lax.fori_loop(0, share, win, 0)


def _make_hist(shift):
    return functools.partial(
        pl.kernel, mesh=_mesh, compiler_params=_params,
        out_type=jax.ShapeDtypeStruct((NW, NBINS), jnp.int32),
        scratch_types=[pltpu.VMEM((NW, 16), jnp.int32),
                       pltpu.VMEM((W,), jnp.int32),
                       pltpu.VMEM((NBINS,), jnp.int32),
                       pltpu.SemaphoreType.DMA],
    )(functools.partial(_hist_body, shift))


def _make_perm(shift, is_final):
    if is_final:
        out_type = jax.ShapeDtypeStruct((N_TOTAL + PAD,), jnp.float32)
        scratch = [pltpu.VMEM((NW, 16), jnp.int32),
                   pltpu.VMEM((NW, NBINS), jnp.int32),
                   pltpu.VMEM((NBINS,), jnp.int32),
                   pltpu.VMEM((W,), jnp.int32),
                   pltpu.VMEM((W,), jnp.float32),
                   pltpu.VMEM((W,), jnp.int32),
                   pltpu.VMEM((W,), jnp.float32),
                   pltpu.SemaphoreType.DMA]
    else:
        out_type = [jax.ShapeDtypeStruct((N_TOTAL + PAD,), jnp.int32),
                    jax.ShapeDtypeStruct((N_TOTAL + PAD,), jnp.float32)]
        scratch = [pltpu.VMEM((NW, 16), jnp.int32),
                   pltpu.VMEM((NW, NBINS), jnp.int32),
                   pltpu.VMEM((NBINS,), jnp.int32),
                   pltpu.VMEM((W,), jnp.int32),
                   pltpu.VMEM((W,), jnp.float32),
                   pltpu.VMEM((W,), jnp.int32),
                   pltpu.SemaphoreType.DMA]
    return functools.partial(
        pl.kernel, mesh=_mesh, compiler_params=_params,
        out_type=out_type, scratch_types=scratch,
    )(functools.partial(_perm_body, shift, is_final))


_hist_k = [_make_hist(s) for s in SHIFTS]
_perm_k = [_make_perm(SHIFTS[0], False), _make_perm(SHIFTS[1], False),
           _make_perm(SHIFTS[2], True)]


def kernel(input, target):
    inp = input.reshape(-1)
    tgt = target.reshape(-1)
    val, kw = _elemwise(inp, tgt)
    counts = _sc_count(tgt)
    g = jax.random.gumbel(jax.random.key(1), (N_TOTAL,), dtype=jnp.float32)
    g_ext = jnp.concatenate([g, jnp.zeros((PAD,), jnp.float32)])
    out_ext, ku0, nv0 = _sc_compact(counts, kw, val, g_ext)
    h0 = _hist_k[0](counts, ku0)
    ku1, nv1 = _perm_k[0](counts, h0, ku0, nv0)
    h1 = _hist_k[1](counts, ku1)
    ku2, nv2 = _perm_k[1](counts, h1, ku1, nv1)
    h2 = _hist_k[2](counts, ku2)
    outn = _perm_k[2](counts, h2, ku2, nv2)
    p_total = jnp.sum(counts[:, 0])
    out = jnp.where(jnp.arange(N_TOTAL) < p_total,
                    out_ext[:N_TOTAL], outn[:N_TOTAL])
    return out


# speedup vs baseline: 1.0074x; 1.0074x over previous
import functools

import jax
import jax.numpy as jnp
from jax import lax
from jax.experimental import pallas as pl
from jax.experimental.pallas import tpu as pltpu
from jax.experimental.pallas import tpu_sc as plsc

MARGIN = 0.0
EPS = 1e-08
SELECT_RATIO = 30

N_TOTAL = 12_800_000
ROWS = 1000
COLS = 12_800
BLK_ROWS = 8

NC = 2
NS = 16
NW = NC * NS
CHUNK = N_TOTAL // NW
W = 8000
NWIN = CHUNK // W
NVREG = W // 16
PAD = NW * W
NBINS = 2048
SHIFTS = (0, 11, 22)

_mesh = plsc.VectorSubcoreMesh(core_axis_name="c", subcore_axis_name="s")
_params = pltpu.CompilerParams(needs_layout_passes=False)


def _elemwise_body(inp_ref, tgt_ref, val_ref, kw_ref):
    x = inp_ref[...]
    t = tgt_ref[...]
    sig = jax.nn.sigmoid(x)
    w = 1.0 - sig
    m = t > MARGIN
    vpos = -jnp.log(jnp.clip(sig, EPS, 1.0 - EPS))
    vneg = -jnp.log(jnp.maximum(w, EPS))
    val_ref[...] = jnp.where(m, vpos, vneg)
    logw = jnp.log(jnp.maximum(w, 1e-30))
    kw_ref[...] = jnp.where(m, 1.0, logw)


def _elemwise(inp_flat, tgt_flat):
    inp2 = inp_flat.reshape(ROWS, COLS)
    tgt2 = tgt_flat.reshape(ROWS, COLS)
    val, kw = pl.pallas_call(
        _elemwise_body,
        grid=(ROWS // BLK_ROWS,),
        in_specs=[
            pl.BlockSpec((BLK_ROWS, COLS), lambda i: (i, 0)),
            pl.BlockSpec((BLK_ROWS, COLS), lambda i: (i, 0)),
        ],
        out_specs=[
            pl.BlockSpec((BLK_ROWS, COLS), lambda i: (i, 0)),
            pl.BlockSpec((BLK_ROWS, COLS), lambda i: (i, 0)),
        ],
        out_shape=[
            jax.ShapeDtypeStruct((ROWS, COLS), jnp.float32),
            jax.ShapeDtypeStruct((ROWS, COLS), jnp.float32),
        ],
    )(inp2, tgt2)
    return val.reshape(-1), kw.reshape(-1)


def _wid():
    return lax.axis_index("s") * NC + lax.axis_index("c")


def _iota16():
    return lax.iota(jnp.int32, 16)


def _splat(x):
    return jnp.full((16,), x, jnp.int32)


@functools.partial(
    pl.kernel, mesh=_mesh, compiler_params=_params,
    out_type=jax.ShapeDtypeStruct((NW, 16), jnp.int32),
    scratch_types=[pltpu.VMEM((W,), jnp.float32),
                   pltpu.VMEM((16,), jnp.int32),
                   pltpu.SemaphoreType.DMA],
)
def _sc_count(tgt_hbm, counts_hbm, buf_v, cnt_v, sem):
    wid = _wid()
    base = wid * CHUNK

    def win(k, acc):
        pltpu.sync_copy(tgt_hbm.at[pl.ds(base + k * W, W)], buf_v)

        def vstep(j, a):
            t = buf_v[pl.ds(j * 16, 16)]
            return a + jnp.where(t > MARGIN, 1, 0).astype(jnp.int32)

        return lax.fori_loop(0, NVREG, vstep, acc)

    acc = lax.fori_loop(0, NWIN, win, jnp.zeros((16,), jnp.int32))
    cnt_v[...] = _splat(jnp.sum(acc))
    pltpu.sync_copy(cnt_v, counts_hbm.at[wid])


def _load_counts(counts_hbm, counts_v, wid):
    pltpu.sync_copy(counts_hbm, counts_v)
    psum = jnp.int32(0)
    total = jnp.int32(0)
    for i in range(NW):
        v = jnp.max(counts_v[i, :])
        psum = psum + jnp.where(i < wid, v, 0)
        total = total + v
    return psum, total


@functools.partial(
    pl.kernel, mesh=_mesh, compiler_params=_params,
    out_type=[jax.ShapeDtypeStruct((N_TOTAL + PAD,), jnp.float32),
              jax.ShapeDtypeStruct((N_TOTAL + PAD,), jnp.int32),
              jax.ShapeDtypeStruct((N_TOTAL + PAD,), jnp.float32)],
    scratch_types=[pltpu.VMEM((NW, 16), jnp.int32),
                   pltpu.VMEM((W,), jnp.float32),
                   pltpu.VMEM((W,), jnp.float32),
                   pltpu.VMEM((W,), jnp.int32),
                   pltpu.VMEM((W,), jnp.int32),
                   pltpu.VMEM((W + 16,), jnp.float32),
                   pltpu.VMEM((W,), jnp.int32),
                   pltpu.VMEM((16,), jnp.int32),
                   pltpu.SemaphoreType.DMA],
)
def _sc_compact(counts_hbm, kw_hbm, val_hbm, g_hbm,
                out_hbm, ku_hbm, nv_hbm,
                counts_v, kwv, valv, oidx_v, nidx_v, gv, kuv, tab_v, sem):
    wid = _wid()
    pos_base, _total = _load_counts(counts_hbm, counts_v, wid)
    neg_base = wid * CHUNK - pos_base
    base = wid * CHUNK
    iota = _iota16()
    dump = N_TOTAL + wid * W
    tab_v[...] = jnp.where(iota == 0, pos_base,
                           jnp.where(iota == 1, neg_base, 0))

    def win(k, c0):
        pltpu.sync_copy(kw_hbm.at[pl.ds(base + k * W, W)], kwv)
        pltpu.sync_copy(val_hbm.at[pl.ds(base + k * W, W)], valv)
        negrun0 = tab_v[...][1]
        a0 = pl.multiple_of(lax.shift_left(
            lax.shift_right_logical(negrun0, 3), 3), 8)
        pltpu.sync_copy(g_hbm.at[pl.ds(a0, W + 16)], gv)

        def vstep(j, c):
            kvec = kwv[pl.ds(j * 16, 16)]
            m = kvec > 0.0
            d = jnp.where(m, 0, 1)
            occ, lastm = plsc.scan_count(d)
            cur = plsc.load_gather(tab_v, [d])
            gidx = cur + occ - 1
            plsc.addupdate_scatter(tab_v, [d], occ, mask=lastm)
            wpos = j * 16 + iota
            oidx_v[pl.ds(j * 16, 16)] = jnp.where(m, gidx, dump + wpos)
            nidx_v[pl.ds(j * 16, 16)] = jnp.where(m, dump + wpos, gidx)
            loc = jnp.where(m, 0, gidx - a0)
            gvec = plsc.load_gather(gv, [loc])
            key = kvec + gvec
            bits = plsc.bitcast(key, jnp.int32)
            ku = jnp.where(bits < 0, bits, bits ^ jnp.int32(0x7FFFFFFF))
            kuv[pl.ds(j * 16, 16)] = ku
            return c

        lax.fori_loop(0, NVREG, vstep, 0)
        cp1 = pltpu.async_copy(valv, out_hbm.at[oidx_v], sem)
        cp2 = pltpu.async_copy(kuv, ku_hbm.at[nidx_v], sem)
        cp3 = pltpu.async_copy(valv, nv_hbm.at[nidx_v], sem)
        cp1.wait()
        cp2.wait()
        cp3.wait()
        return c0

    lax.fori_loop(0, NWIN, win, 0)


def _hist_body(shift, counts_hbm, ku_hbm, hist_hbm, counts_v, kuv, hist_v, sem):
    wid = _wid()
    _pb, total = _load_counts(counts_hbm, counts_v, wid)
    nneg = N_TOTAL - total
    nwin_total = (nneg + (W - 1)) // W
    share = (nwin_total + (NW - 1)) // NW
    iota = _iota16()

    def zstep(j, c):
        hist_v[pl.ds(j * 16, 16)] = jnp.zeros((16,), jnp.int32)
        return c

    lax.fori_loop(0, NBINS // 16, zstep, 0)

    def win(t, c):
        gbase = (wid * share + t) * W

        @pl.when(gbase < nneg)
        def _():
            pltpu.sync_copy(ku_hbm.at[pl.ds(gbase, W)], kuv)

            def vstep(j, c2):
                ku = kuv[pl.ds(j * 16, 16)]
                gpos = gbase + j * 16 + iota
                m = gpos < nneg
                d = lax.shift_right_logical(ku, shift) & (NBINS - 1)
                occ, lastm = plsc.scan_count(d, mask=m)
                plsc.addupdate_scatter(hist_v, [d], occ, mask=lastm)
                return c2

            lax.fori_loop(0, NVREG, vstep, 0)

        return c

    lax.fori_loop(0, share, win, 0)
    pltpu.sync_copy(hist_v, hist_hbm.at[wid])


def _perm_body(shift, is_final,
               counts_hbm, hist_hbm, ku_hbm, nv_hbm, *refs):
    if is_final:
        (outn_hbm, counts_v, histf_v, offs_v, kuv, vav, didx_v, dat_v,
         sem) = refs
    else:
        (outk_hbm, outv_hbm, counts_v, histf_v, offs_v, kuv, vav, didx_v,
         sem) = refs
    wid = _wid()
    pos_base, total = _load_counts(counts_hbm, counts_v, wid)
    nneg = N_TOTAL - total
    nsamp = jnp.minimum(SELECT_RATIO * total, nneg)
    nwin_total = (nneg + (W - 1)) // W
    share = (nwin_total + (NW - 1)) // NW
    iota = _iota16()
    dump = N_TOTAL + wid * W

    pltpu.sync_copy(hist_hbm, histf_v)

    def dgroup(jg, runtot):
        zero = jnp.zeros((16,), jnp.int32)

        def wrow(i, cp):
            cs, ps = cp
            row = histf_v[i, pl.ds(jg * 16, 16)]
            return (cs + row, ps + jnp.where(_splat(i) < wid, row, 0))

        colsum, presum = lax.fori_loop(0, NW, wrow, (zero, zero))
        excl = plsc.cumsum(colsum) - colsum
        offs_v[pl.ds(jg * 16, 16)] = runtot + excl + presum
        return runtot + jnp.sum(colsum)

    lax.fori_loop(0, NBINS // 16, dgroup, jnp.int32(0))

    def win(t, c):
        gbase = (wid * share + t) * W

        @pl.when(gbase < nneg)
        def _():
            pltpu.sync_copy(ku_hbm.at[pl.ds(gbase, W)], kuv)
            pltpu.sync_copy(nv_hbm.at[pl.ds(gbase, W)], vav)

            def vstep(j, c2):
                ku = kuv[pl.ds(j * 16, 16)]
                gpos = gbase + j * 16 + iota
                m = gpos < nneg
                d = lax.shift_right_logical(ku, shift) & (NBINS - 1)
                occ, lastm = plsc.scan_count(d, mask=m)
                cur = plsc.load_gather(offs_v, [d])
                pos = cur + occ - 1
                plsc.addupdate_scatter(offs_v, [d], occ, mask=lastm)
                wpos = j * 16 + iota
                if is_final:
                    didx_v[pl.ds(j * 16, 16)] = jnp.where(
                        m, total + pos, dump + wpos)
                    va = vav[pl.ds(j * 16, 16)]
                    dat_v[pl.ds(j * 16, 16)] = jnp.where(
                        jnp.logical_and(m, pos < nsamp), va, 0.0)
                else:
                    didx_v[pl.ds(j * 16, 16)] = jnp.where(m, pos, dump + wpos)
                return c2

            lax.fori_loop(0, NVREG, vstep, 0)
            if is_final:
                pltpu.async_copy(dat_v, outn_hbm.at[didx_v], sem).wait()
            else:
                cp1 = pltpu.async_copy(kuv, outk_hbm.at[didx_v], sem)
                cp2 = pltpu.async_copy(vav, outv_hbm.at[didx_v], sem)
                cp1.wait()
                cp2.wait()

        return c

    lax.fori_loop(0, share, win, 0)


def _make_hist(shift):
    return functools.partial(
        pl.kernel, mesh=_mesh, compiler_params=_params,
        out_type=jax.ShapeDtypeStruct((NW, NBINS), jnp.int32),
        scratch_types=[pltpu.VMEM((NW, 16), jnp.int32),
                       pltpu.VMEM((W,), jnp.int32),
                       pltpu.VMEM((NBINS,), jnp.int32),
                       pltpu.SemaphoreType.DMA],
    )(functools.partial(_hist_body, shift))


def _make_perm(shift, is_final):
    if is_final:
        out_type = jax.ShapeDtypeStruct((N_TOTAL + PAD,), jnp.float32)
        scratch = [pltpu.VMEM((NW, 16), jnp.int32),
                   pltpu.VMEM((NW, NBINS), jnp.int32),
                   pltpu.VMEM((NBINS,), jnp.int32),
                   pltpu.VMEM((W,), jnp.int32),
                   pltpu.VMEM((W,), jnp.float32),
                   pltpu.VMEM((W,), jnp.int32),
                   pltpu.VMEM((W,), jnp.float32),
                   pltpu.SemaphoreType.DMA]
    else:
        out_type = [jax.ShapeDtypeStruct((N_TOTAL + PAD,), jnp.int32),
                    jax.ShapeDtypeStruct((N_TOTAL + PAD,), jnp.float32)]
        scratch = [pltpu.VMEM((NW, 16), jnp.int32),
                   pltpu.VMEM((NW, NBINS), jnp.int32),
                   pltpu.VMEM((NBINS,), jnp.int32),
                   pltpu.VMEM((W,), jnp.int32),
                   pltpu.VMEM((W,), jnp.float32),
                   pltpu.VMEM((W,), jnp.int32),
                   pltpu.SemaphoreType.DMA]
    return functools.partial(
        pl.kernel, mesh=_mesh, compiler_params=_params,
        out_type=out_type, scratch_types=scratch,
    )(functools.partial(_perm_body, shift, is_final))


_hist_k = [_make_hist(s) for s in SHIFTS]
_perm_k = [_make_perm(SHIFTS[0], False), _make_perm(SHIFTS[1], False),
           _make_perm(SHIFTS[2], True)]


def kernel(input, target):
    inp = input.reshape(-1)
    tgt = target.reshape(-1)
    val, kw = _elemwise(inp, tgt)
    counts = _sc_count(tgt)
    g = jax.random.gumbel(jax.random.key(1), (N_TOTAL,), dtype=jnp.float32)
    g_ext = jnp.concatenate([g, jnp.zeros((PAD,), jnp.float32)])
    out_ext, ku0, nv0 = _sc_compact(counts, kw, val, g_ext)
    h0 = _hist_k[0](counts, ku0)
    ku1, nv1 = _perm_k[0](counts, h0, ku0, nv0)
    h1 = _hist_k[1](counts, ku1)
    ku2, nv2 = _perm_k[1](counts, h1, ku1, nv1)
    h2 = _hist_k[2](counts, ku2)
    outn = _perm_k[2](counts, h2, ku2, nv2)
    p_total = jnp.sum(counts[:, 0])
    out = jnp.where(jnp.arange(N_TOTAL) < p_total,
                    out_ext[:N_TOTAL], outn[:N_TOTAL])
    return out


# speedup vs baseline: 1.3830x; 1.3728x over previous
import functools

import jax
import jax.numpy as jnp
from jax import lax
from jax.experimental import pallas as pl
from jax.experimental.pallas import tpu as pltpu
from jax.experimental.pallas import tpu_sc as plsc

MARGIN = 0.0
EPS = 1e-08
SELECT_RATIO = 30

N_TOTAL = 12_800_000
ROWS = 1000
COLS = 12_800
BLK_ROWS = 8

NC = 2
NS = 16
NW = NC * NS
CHUNK = N_TOTAL // NW
W = 8000
NWIN = CHUNK // W
NVREG = W // 16
PAD = NW * W
NBINS = 2048
SHIFTS = (0, 11, 22)

_mesh = plsc.VectorSubcoreMesh(core_axis_name="c", subcore_axis_name="s")
_params = pltpu.CompilerParams(needs_layout_passes=False)


def _elemwise_body(inp_ref, tgt_ref, val_ref, kw_ref):
    x = inp_ref[...]
    t = tgt_ref[...]
    sig = jax.nn.sigmoid(x)
    w = 1.0 - sig
    m = t > MARGIN
    vpos = -jnp.log(jnp.clip(sig, EPS, 1.0 - EPS))
    vneg = -jnp.log(jnp.maximum(w, EPS))
    val_ref[...] = jnp.where(m, vpos, vneg)
    logw = jnp.log(jnp.maximum(w, 1e-30))
    kw_ref[...] = jnp.where(m, 1.0, logw)


def _elemwise(inp_flat, tgt_flat):
    inp2 = inp_flat.reshape(ROWS, COLS)
    tgt2 = tgt_flat.reshape(ROWS, COLS)
    val, kw = pl.pallas_call(
        _elemwise_body,
        grid=(ROWS // BLK_ROWS,),
        in_specs=[
            pl.BlockSpec((BLK_ROWS, COLS), lambda i: (i, 0)),
            pl.BlockSpec((BLK_ROWS, COLS), lambda i: (i, 0)),
        ],
        out_specs=[
            pl.BlockSpec((BLK_ROWS, COLS), lambda i: (i, 0)),
            pl.BlockSpec((BLK_ROWS, COLS), lambda i: (i, 0)),
        ],
        out_shape=[
            jax.ShapeDtypeStruct((ROWS, COLS), jnp.float32),
            jax.ShapeDtypeStruct((ROWS, COLS), jnp.float32),
        ],
    )(inp2, tgt2)
    return val.reshape(-1), kw.reshape(-1)


def _wid():
    return lax.axis_index("s") * NC + lax.axis_index("c")


def _iota16():
    return lax.iota(jnp.int32, 16)


def _splat(x):
    return jnp.full((16,), x, jnp.int32)


@functools.partial(
    pl.kernel, mesh=_mesh, compiler_params=_params,
    out_type=jax.ShapeDtypeStruct((NW, 16), jnp.int32),
    scratch_types=[pltpu.VMEM((W,), jnp.float32),
                   pltpu.VMEM((16,), jnp.int32),
                   pltpu.SemaphoreType.DMA],
)
def _sc_count(tgt_hbm, counts_hbm, buf_v, cnt_v, sem):
    wid = _wid()
    base = wid * CHUNK

    def win(k, acc):
        pltpu.sync_copy(tgt_hbm.at[pl.ds(base + k * W, W)], buf_v)

        def vstep(j, a):
            t = buf_v[pl.ds(j * 16, 16)]
            return a + jnp.where(t > MARGIN, 1, 0).astype(jnp.int32)

        return lax.fori_loop(0, NVREG, vstep, acc)

    acc = lax.fori_loop(0, NWIN, win, jnp.zeros((16,), jnp.int32))
    cnt_v[...] = _splat(jnp.sum(acc))
    pltpu.sync_copy(cnt_v, counts_hbm.at[wid])


def _load_counts(counts_hbm, counts_v, wid):
    pltpu.sync_copy(counts_hbm, counts_v)
    psum = jnp.int32(0)
    total = jnp.int32(0)
    for i in range(NW):
        v = jnp.max(counts_v[i, :])
        psum = psum + jnp.where(i < wid, v, 0)
        total = total + v
    return psum, total


@functools.partial(
    pl.kernel, mesh=_mesh, compiler_params=_params,
    out_type=[jax.ShapeDtypeStruct((N_TOTAL + PAD,), jnp.float32),
              jax.ShapeDtypeStruct((2 * (N_TOTAL + PAD),), jnp.int32)],
    scratch_types=[pltpu.VMEM((NW, 16), jnp.int32),
                   pltpu.VMEM((W,), jnp.float32),
                   pltpu.VMEM((W,), jnp.float32),
                   pltpu.VMEM((2 * W,), jnp.float32),
                   pltpu.VMEM((2 * W,), jnp.float32),
                   pltpu.VMEM((2 * W,), jnp.float32),
                   pltpu.VMEM((2 * W,), jnp.int32),
                   pltpu.VMEM((2 * W,), jnp.int32),
                   pltpu.VMEM((W,), jnp.int32),
                   pltpu.VMEM((W + 16,), jnp.float32),
                   pltpu.SemaphoreType.DMA],
)
def _sc_compact(counts_hbm, kw_hbm, val_hbm, g_hbm,
                out_hbm, kv_hbm,
                counts_v, kwv, valv, pstag, nkstag, nvstag,
                pairb, idx2b, idxb, gv, sem):
    wid = _wid()
    pos_base, _total = _load_counts(counts_hbm, counts_v, wid)
    base = wid * CHUNK
    iota = _iota16()
    dump = N_TOTAL + wid * W

    def flush_pos(wp, pflushed, partial):
        def istep(j, c):
            bp = j * 16 + iota
            tgt = pos_base + pflushed + bp
            if partial:
                tgt = jnp.where(bp < wp, tgt, dump + bp)
            idxb[pl.ds(j * 16, 16)] = tgt
            return c

        lax.fori_loop(0, NVREG, istep, 0)
        pltpu.async_copy(pstag.at[pl.ds(0, W)], out_hbm.at[idxb], sem).wait()
        if partial:
            return wp, pflushed

        def shift(j, c):
            v = pstag[pl.ds(W + j * 16, 16)]
            pstag[pl.ds(j * 16, 16)] = v
            return c

        lax.fori_loop(0, (wp - W + 15) // 16, shift, 0)
        return wp - W, pflushed + W

    def flush_neg(wn, nflushed, partial):
        a0 = pl.multiple_of(
            lax.shift_left(lax.shift_right_logical(nflushed, 3), 3), 8)
        sh = nflushed - a0
        pltpu.sync_copy(g_hbm.at[pl.ds(a0, W + 16)], gv)

        def istep(j, c):
            bp = j * 16 + iota
            kwc = nkstag[pl.ds(j * 16, 16)]
            vvc = nvstag[pl.ds(j * 16, 16)]
            gvec = gv[pl.ds(sh + j * 16, 16)]
            key = kwc + gvec
            bits = plsc.bitcast(key, jnp.int32)
            ku = jnp.where(bits < 0, bits, bits ^ jnp.int32(0x7FFFFFFF))
            plsc.store_scatter(pairb, [2 * bp], ku)
            plsc.store_scatter(pairb, [2 * bp + 1], plsc.bitcast(vvc, jnp.int32))
            tgt = nflushed + bp
            if partial:
                tgt = jnp.where(bp < wn, tgt, dump + bp)
            plsc.store_scatter(idx2b, [2 * bp], 2 * tgt)
            plsc.store_scatter(idx2b, [2 * bp + 1], 2 * tgt + 1)
            return c

        lax.fori_loop(0, NVREG, istep, 0)
        pltpu.async_copy(pairb, kv_hbm.at[idx2b], sem).wait()
        if partial:
            return wn, nflushed

        def shift(j, c):
            nkstag[pl.ds(j * 16, 16)] = nkstag[pl.ds(W + j * 16, 16)]
            nvstag[pl.ds(j * 16, 16)] = nvstag[pl.ds(W + j * 16, 16)]
            return c

        lax.fori_loop(0, (wn - W + 15) // 16, shift, 0)
        return wn - W, nflushed + W

    def win(k, carry):
        wp, wn, pfl, nfl = carry
        pltpu.sync_copy(kw_hbm.at[pl.ds(base + k * W, W)], kwv)
        pltpu.sync_copy(val_hbm.at[pl.ds(base + k * W, W)], valv)

        def vstep(j, c):
            wp2, wn2 = c
            kvec = kwv[pl.ds(j * 16, 16)]
            vvec = valv[pl.ds(j * 16, 16)]
            m = kvec > 0.0
            mi = m.astype(jnp.int32)
            pc = plsc.cumsum(mi)
            nc = (iota + 1) - pc
            plsc.store_scatter(pstag, [wp2 + pc - 1], vvec, mask=m)
            nm = jnp.logical_not(m)
            plsc.store_scatter(nkstag, [wn2 + nc - 1], kvec, mask=nm)
            plsc.store_scatter(nvstag, [wn2 + nc - 1], vvec, mask=nm)
            cntp = jnp.sum(mi)
            return (wp2 + cntp, wn2 + 16 - cntp)

        wp, wn = lax.fori_loop(0, NVREG, vstep, (wp, wn))

        wp, pfl = lax.cond(wp >= W,
                           lambda: flush_pos(wp, pfl, False),
                           lambda: (wp, pfl))
        wn, nfl = lax.cond(wn >= W,
                           lambda: flush_neg(wn, nfl, False),
                           lambda: (wn, nfl))
        return (wp, wn, pfl, nfl)

    neg_base = wid * CHUNK - pos_base
    wp, wn, pfl, nfl = lax.fori_loop(
        0, NWIN, win, (jnp.int32(0), jnp.int32(0), jnp.int32(0), neg_base))
    flush_pos(wp, pfl, True)
    flush_neg(wn, nfl, True)


def _hist_body(shift, counts_hbm, kv_hbm, hist_hbm,
               counts_v, kuv1d, hist_v, sem):
    wid = _wid()
    _pb, total = _load_counts(counts_hbm, counts_v, wid)
    nneg = N_TOTAL - total
    nwin_total = (nneg + (W - 1)) // W
    share = (nwin_total + (NW - 1)) // NW
    iota = _iota16()

    def zstep(j, c):
        hist_v[pl.ds(j * 16, 16)] = jnp.zeros((16,), jnp.int32)
        return c

    lax.fori_loop(0, NBINS // 16, zstep, 0)

    def win(t, c):
        gbase = (wid * share + t) * W

        @pl.when(gbase < nneg)
        def _():
            pltpu.sync_copy(kv_hbm.at[pl.ds(2 * gbase, 2 * W)], kuv1d)

            def vstep(j, c2):
                ku = plsc.load_gather(kuv1d, [2 * (j * 16 + iota)])
                gpos = gbase + j * 16 + iota
                m = gpos < nneg
                d = lax.shift_right_logical(ku, shift) & (NBINS - 1)
                occ, lastm = plsc.scan_count(d, mask=m)
                plsc.addupdate_scatter(hist_v, [d], occ, mask=lastm)
                return c2

            lax.fori_loop(0, NVREG, vstep, 0)

        return c

    lax.fori_loop(0, share, win, 0)
    pltpu.sync_copy(hist_v, hist_hbm.at[wid])


def _perm_body(shift, is_final,
               counts_hbm, hist_hbm, kv_hbm, *refs):
    if is_final:
        (outn_hbm, counts_v, histf_v, offs_v, kuv1d, didx_v, dat_v,
         sem) = refs
    else:
        (outkv_hbm, counts_v, histf_v, offs_v, kuv1d, didx_v,
         sem) = refs
    wid = _wid()
    pos_base, total = _load_counts(counts_hbm, counts_v, wid)
    nneg = N_TOTAL - total
    nsamp = jnp.minimum(SELECT_RATIO * total, nneg)
    nwin_total = (nneg + (W - 1)) // W
    share = (nwin_total + (NW - 1)) // NW
    iota = _iota16()
    dump = N_TOTAL + wid * W

    pltpu.sync_copy(hist_hbm, histf_v)

    def dgroup(jg, runtot):
        zero = jnp.zeros((16,), jnp.int32)

        def wrow(i, cp):
            cs, ps = cp
            row = histf_v[i, pl.ds(jg * 16, 16)]
            return (cs + row, ps + jnp.where(_splat(i) < wid, row, 0))

        colsum, presum = lax.fori_loop(0, NW, wrow, (zero, zero))
        excl = plsc.cumsum(colsum) - colsum
        offs_v[pl.ds(jg * 16, 16)] = runtot + excl + presum
        return runtot + jnp.sum(colsum)

    lax.fori_loop(0, NBINS // 16, dgroup, jnp.int32(0))

    def win(t, c):
        gbase = (wid * share + t) * W

        @pl.when(gbase < nneg)
        def _():
            pltpu.sync_copy(kv_hbm.at[pl.ds(2 * gbase, 2 * W)], kuv1d)

            def vstep(j, c2):
                bp = j * 16 + iota
                ku = plsc.load_gather(kuv1d, [2 * bp])
                gpos = gbase + bp
                m = gpos < nneg
                d = lax.shift_right_logical(ku, shift) & (NBINS - 1)
                occ, lastm = plsc.scan_count(d, mask=m)
                cur = plsc.load_gather(offs_v, [d])
                pos = cur + occ - 1
                plsc.addupdate_scatter(offs_v, [d], occ, mask=lastm)
                if is_final:
                    didx_v[pl.ds(j * 16, 16)] = jnp.where(
                        m, total + pos, dump + bp)
                    va = plsc.bitcast(
                        plsc.load_gather(kuv1d, [2 * bp + 1]), jnp.float32)
                    dat_v[pl.ds(j * 16, 16)] = jnp.where(
                        jnp.logical_and(m, pos < nsamp), va, 0.0)
                else:
                    tgt = jnp.where(m, pos, dump + bp)
                    plsc.store_scatter(didx_v, [2 * bp], 2 * tgt)
                    plsc.store_scatter(didx_v, [2 * bp + 1], 2 * tgt + 1)
                return c2

            lax.fori_loop(0, NVREG, vstep, 0)
            if is_final:
                pltpu.async_copy(dat_v, outn_hbm.at[didx_v], sem).wait()
            else:
                pltpu.async_copy(kuv1d, outkv_hbm.at[didx_v], sem).wait()

        return c

    lax.fori_loop(0, share, win, 0)


def _make_hist(shift):
    return functools.partial(
        pl.kernel, mesh=_mesh, compiler_params=_params,
        out_type=jax.ShapeDtypeStruct((NW, NBINS), jnp.int32),
        scratch_types=[pltpu.VMEM((NW, 16), jnp.int32),
                       pltpu.VMEM((2 * W,), jnp.int32),
                       pltpu.VMEM((NBINS,), jnp.int32),
                       pltpu.SemaphoreType.DMA],
    )(functools.partial(_hist_body, shift))


def _make_perm(shift, is_final):
    if is_final:
        out_type = jax.ShapeDtypeStruct((N_TOTAL + PAD,), jnp.float32)
        scratch = [pltpu.VMEM((NW, 16), jnp.int32),
                   pltpu.VMEM((NW, NBINS), jnp.int32),
                   pltpu.VMEM((NBINS,), jnp.int32),
                   pltpu.VMEM((2 * W,), jnp.int32),
                   pltpu.VMEM((W,), jnp.int32),
                   pltpu.VMEM((W,), jnp.float32),
                   pltpu.SemaphoreType.DMA]
    else:
        out_type = jax.ShapeDtypeStruct((2 * (N_TOTAL + PAD),), jnp.int32)
        scratch = [pltpu.VMEM((NW, 16), jnp.int32),
                   pltpu.VMEM((NW, NBINS), jnp.int32),
                   pltpu.VMEM((NBINS,), jnp.int32),
                   pltpu.VMEM((2 * W,), jnp.int32),
                   pltpu.VMEM((2 * W,), jnp.int32),
                   pltpu.SemaphoreType.DMA]
    return functools.partial(
        pl.kernel, mesh=_mesh, compiler_params=_params,
        out_type=out_type, scratch_types=scratch,
    )(functools.partial(_perm_body, shift, is_final))


_hist_k = [_make_hist(s) for s in SHIFTS]
_perm_k = [_make_perm(SHIFTS[0], False), _make_perm(SHIFTS[1], False),
           _make_perm(SHIFTS[2], True)]


def kernel(input, target):
    inp = input.reshape(-1)
    tgt = target.reshape(-1)
    val, kw = _elemwise(inp, tgt)
    counts = _sc_count(tgt)
    g = jax.random.gumbel(jax.random.key(1), (N_TOTAL,), dtype=jnp.float32)
    g_ext = jnp.concatenate([g, jnp.zeros((PAD,), jnp.float32)])
    out_ext, kv0 = _sc_compact(counts, kw, val, g_ext)
    h0 = _hist_k[0](counts, kv0)
    kv1 = _perm_k[0](counts, h0, kv0)
    h1 = _hist_k[1](counts, kv1)
    kv2 = _perm_k[1](counts, h1, kv1)
    h2 = _hist_k[2](counts, kv2)
    outn = _perm_k[2](counts, h2, kv2)
    p_total = jnp.sum(counts[:, 0])
    out = jnp.where(jnp.arange(N_TOTAL) < p_total,
                    out_ext[:N_TOTAL], outn[:N_TOTAL])
    return out
